# speedup vs baseline: 5.2252x; 5.2252x over previous
import functools
import math

import jax
import jax.numpy as jnp
from jax import lax
from jax.experimental import pallas as pl
from jax.experimental.pallas import tpu as pltpu
from jax.experimental.pallas import tpu_sc as plsc

D = 128
SCALE = math.sqrt(128.0)

NUM_CORES = 2
NUM_SUBCORES = 16
NW = NUM_CORES * NUM_SUBCORES

CHUNK = 128
NBUF = 5
NCHUNK = 50
GROUPS = NCHUNK // NBUF


def _scale_body(t_ref, o_ref):
    o_ref[...] = t_ref[...] * SCALE


def _scale_table(table):
    rows, d = table.shape
    blk = 2000
    return pl.pallas_call(
        _scale_body,
        grid=(rows // blk,),
        in_specs=[pl.BlockSpec((blk, d), lambda i: (i, 0))],
        out_specs=pl.BlockSpec((blk, d), lambda i: (i, 0)),
        out_shape=jax.ShapeDtypeStruct((rows, d), jnp.float32),
    )(table)


def _gather_body(idx_hbm, table_hbm, out_hbm, idx_v, bufs, *sems):
    sg = sems[:NBUF]
    so = sems[NBUF:]
    wid = lax.axis_index("s") * NUM_CORES + lax.axis_index("c")
    base = wid * (NCHUNK * CHUNK)

    pltpu.sync_copy(idx_hbm.at[wid], idx_v)

    for b in range(NBUF):
        pltpu.async_copy(table_hbm.at[idx_v.at[b]], bufs.at[b], sg[b])

    def group(g, carry):
        for b in range(NBUF):
            c = g * NBUF + b
            pltpu.make_async_copy(
                table_hbm.at[idx_v.at[c]], bufs.at[b], sg[b]).wait()
            pltpu.async_copy(
                bufs.at[b], out_hbm.at[pl.ds(base + c * CHUNK, CHUNK)], so[b])

            @pl.when(g < GROUPS - 1)
            def _():
                pltpu.make_async_copy(
                    bufs.at[b], out_hbm.at[pl.ds(base, CHUNK)], so[b]).wait()
                pltpu.async_copy(
                    table_hbm.at[idx_v.at[c + NBUF]], bufs.at[b], sg[b])
        return carry

    lax.fori_loop(0, GROUPS, group, 0)

    for b in range(NBUF):
        pltpu.make_async_copy(
            bufs.at[b], out_hbm.at[pl.ds(base, CHUNK)], so[b]).wait()


@functools.partial(
    pl.kernel,
    out_type=jax.ShapeDtypeStruct((NW * NCHUNK * CHUNK, D), jnp.float32),
    mesh=plsc.VectorSubcoreMesh(core_axis_name="c", subcore_axis_name="s"),
    scratch_types=[
        pltpu.VMEM((NCHUNK, CHUNK), jnp.int32),
        pltpu.VMEM((NBUF, CHUNK, D), jnp.float32),
    ] + [pltpu.SemaphoreType.DMA] * (2 * NBUF),
)
def _gather(idx_hbm, table_hbm, out_hbm, idx_v, bufs, *sems):
    _gather_body(idx_hbm, table_hbm, out_hbm, idx_v, bufs, *sems)


def kernel(input_ids, embedding_weight):
    b, s = input_ids.shape
    scaled = _scale_table(embedding_weight)
    idx = input_ids.reshape(NW, NCHUNK, CHUNK).astype(jnp.int32)
    out = _gather(idx, scaled)
    return out.reshape(b, s, D)


# speedup vs baseline: 7.9361x; 1.5188x over previous
import functools
import math

import jax
import jax.numpy as jnp
from jax import lax
from jax.experimental import pallas as pl
from jax.experimental.pallas import tpu as pltpu
from jax.experimental.pallas import tpu_sc as plsc

D = 128
SCALE = math.sqrt(128.0)

NUM_CORES = 2
NUM_SUBCORES = 16
NW = NUM_CORES * NUM_SUBCORES

CHUNK = 128
NBUF = 5
NCHUNK = 50
GROUPS = NCHUNK // NBUF


def _gather_body(idx_hbm, table_hbm, out_hbm, idx_v, bufs, *sems):
    sg = sems[:NBUF]
    so = sems[NBUF:]
    wid = lax.axis_index("s") * NUM_CORES + lax.axis_index("c")
    base = wid * (NCHUNK * CHUNK)

    pltpu.sync_copy(idx_hbm.at[wid], idx_v)

    for b in range(NBUF):
        pltpu.async_copy(table_hbm.at[idx_v.at[b]], bufs.at[b], sg[b])

    def group(g, carry):
        for b in range(NBUF):
            c = g * NBUF + b
            pltpu.make_async_copy(
                table_hbm.at[idx_v.at[c]], bufs.at[b], sg[b]).wait()

            def _scale_row(r, _b=b):
                for j in range(D // 16):
                    sl = (_b, r, pl.ds(16 * j, 16))
                    bufs[sl] = bufs[sl] * SCALE

            plsc.parallel_loop(0, CHUNK, unroll=4)(_scale_row)

            pltpu.async_copy(
                bufs.at[b], out_hbm.at[pl.ds(base + c * CHUNK, CHUNK)], so[b])

            @pl.when(g < GROUPS - 1)
            def _():
                pltpu.make_async_copy(
                    bufs.at[b], out_hbm.at[pl.ds(base, CHUNK)], so[b]).wait()
                pltpu.async_copy(
                    table_hbm.at[idx_v.at[c + NBUF]], bufs.at[b], sg[b])
        return carry

    lax.fori_loop(0, GROUPS, group, 0)

    for b in range(NBUF):
        pltpu.make_async_copy(
            bufs.at[b], out_hbm.at[pl.ds(base, CHUNK)], so[b]).wait()


@functools.partial(
    pl.kernel,
    out_type=jax.ShapeDtypeStruct((NW * NCHUNK * CHUNK, D), jnp.float32),
    mesh=plsc.VectorSubcoreMesh(core_axis_name="c", subcore_axis_name="s"),
    scratch_types=[
        pltpu.VMEM((NCHUNK, CHUNK), jnp.int32),
        pltpu.VMEM((NBUF, CHUNK, D), jnp.float32),
    ] + [pltpu.SemaphoreType.DMA] * (2 * NBUF),
)
def _gather(idx_hbm, table_hbm, out_hbm, idx_v, bufs, *sems):
    _gather_body(idx_hbm, table_hbm, out_hbm, idx_v, bufs, *sems)


def kernel(input_ids, embedding_weight):
    b, s = input_ids.shape
    idx = input_ids.reshape(NW, NCHUNK, CHUNK).astype(jnp.int32)
    out = _gather(idx, embedding_weight)
    return out.reshape(b, s, D)
